# baseline (speedup 1.0000x reference)
import functools

import jax
import jax.numpy as jnp
from jax import lax
from jax.experimental import pallas as pl
from jax.experimental.pallas import tpu as pltpu
from jax.experimental.pallas import tpu_sc as plsc

IN_CH = 64
LAT = 64
K_EMB = 8192
EDIM = 64
L_OUT = 1025
L_PAD = 1056
N_B = 8
R_TOT = N_B * L_PAD
R_TILE = 256
K_TILE = 1024
_PREC = lax.Precision.DEFAULT


def _csq_body(cb_ref, csq_ref):
    cbf = cb_ref[...]
    ones = jnp.ones((1, EDIM), dtype=jnp.float32)
    csq_ref[...] = lax.dot_general(ones, cbf * cbf, (((1,), (1,)), ((), ())),
                                   preferred_element_type=jnp.float32,
                                   precision=lax.Precision.HIGHEST)


def _enc_vq_body(e0_ref, o0_ref, e1_ref, o1_ref, w_ref, bc_ref, cb_ref,
                 csq_ref, z_ref, idx_ref):
    zt = jnp.dot(e0_ref[...], w_ref[0], preferred_element_type=jnp.float32,
                 precision=_PREC)
    zt += jnp.dot(o0_ref[...], w_ref[1], preferred_element_type=jnp.float32,
                  precision=_PREC)
    zt += jnp.dot(e1_ref[...], w_ref[2], preferred_element_type=jnp.float32,
                  precision=_PREC)
    zt += jnp.dot(o1_ref[...], w_ref[3], preferred_element_type=jnp.float32,
                  precision=_PREC)
    zt += bc_ref[...]
    z_ref[...] = zt

    zt2 = zt * -2.0
    iota = lax.broadcasted_iota(jnp.int32, (R_TILE, K_TILE), 1).astype(jnp.float32)
    big = jnp.float32(1e9)
    best = jnp.full((R_TILE, 1), jnp.inf, dtype=jnp.float32)
    bidx = jnp.zeros((R_TILE, 1), dtype=jnp.float32)
    for kc in range(K_EMB // K_TILE):
        cbk = cb_ref[kc * K_TILE:(kc + 1) * K_TILE, :]
        d = lax.dot_general(zt2, cbk, (((1,), (1,)), ((), ())),
                            preferred_element_type=jnp.float32,
                            precision=_PREC)
        d = d + csq_ref[:, kc * K_TILE:(kc + 1) * K_TILE]
        m = jnp.min(d, axis=1, keepdims=True)
        sel = jnp.where(d == m, iota, big)
        a = jnp.min(sel, axis=1, keepdims=True) + jnp.float32(kc * K_TILE)
        upd = m < best
        best = jnp.where(upd, m, best)
        bidx = jnp.where(upd, a, bidx)
    idx_ref[...] = bidx.astype(jnp.int32)


def _dec_body(zq0_ref, zq1_ref, m_ref, bt_ref, xe_ref, xo_ref):
    a = zq0_ref[...]
    b = zq1_ref[...]
    bt = bt_ref[...]
    xe_ref[...] = (jnp.dot(b, m_ref[0], preferred_element_type=jnp.float32,
                           precision=_PREC)
                   + jnp.dot(a, m_ref[2], preferred_element_type=jnp.float32,
                             precision=_PREC) + bt)
    xo_ref[...] = (jnp.dot(b, m_ref[1], preferred_element_type=jnp.float32,
                           precision=_PREC)
                   + jnp.dot(a, m_ref[3], preferred_element_type=jnp.float32,
                             precision=_PREC) + bt)


def _sc_gather(idx, table):
    info = plsc.get_sparse_core_info()
    nw = info.num_cores * info.num_subcores
    b_per_w = R_TOT // nw
    dpad = 128

    table_p = jnp.pad(table, ((0, 0), (0, dpad - EDIM)))
    mesh = plsc.VectorSubcoreMesh(core_axis_name="c", subcore_axis_name="s")

    @functools.partial(
        pl.kernel, mesh=mesh,
        out_type=jax.ShapeDtypeStruct((R_TOT, dpad), jnp.float32),
        scratch_types=[
            pltpu.VMEM((b_per_w,), jnp.int32),
            pltpu.VMEM((b_per_w, dpad), jnp.float32),
            pltpu.SemaphoreType.DMA,
        ],
    )
    def gather_k(idx_hbm, table_hbm, out_hbm, idx_v, rows_v, sem):
        wid = lax.axis_index("s") * info.num_cores + lax.axis_index("c")
        base = wid * b_per_w
        pltpu.sync_copy(idx_hbm.at[pl.ds(base, b_per_w)], idx_v)
        pltpu.async_copy(table_hbm.at[idx_v], rows_v, sem).wait()
        pltpu.sync_copy(rows_v, out_hbm.at[pl.ds(base, b_per_w)])

    return gather_k(idx, table_p)[:, :EDIM]


def kernel(x, Wc, bc, codebook, Wt, bt):
    n, c, l_in = x.shape

    xT = jnp.transpose(x, (0, 2, 1))
    xp = jnp.pad(xT, ((0, 0), (2, 2 * L_PAD + 2 - l_in - 2), (0, 0)))
    e0 = xp[:, 0:2 * L_PAD:2].reshape(R_TOT, c)
    o0 = xp[:, 1:2 * L_PAD + 1:2].reshape(R_TOT, c)
    e1 = xp[:, 2:2 * L_PAD + 2:2].reshape(R_TOT, c)
    o1 = xp[:, 3::2].reshape(R_TOT, c)
    wstack = jnp.transpose(Wc, (2, 1, 0))
    bc2 = bc[None, :]

    csq = pl.pallas_call(
        _csq_body,
        out_shape=jax.ShapeDtypeStruct((1, K_EMB), jnp.float32),
    )(codebook)

    grid = R_TOT // R_TILE
    row_spec = pl.BlockSpec((R_TILE, c), lambda i: (i, 0))
    full3 = pl.BlockSpec((4, c, c), lambda i: (0, 0, 0))
    z_rows, idx2d = pl.pallas_call(
        _enc_vq_body,
        grid=(grid,),
        in_specs=[row_spec, row_spec, row_spec, row_spec, full3,
                  pl.BlockSpec((1, c), lambda i: (0, 0)),
                  pl.BlockSpec((K_EMB, EDIM), lambda i: (0, 0)),
                  pl.BlockSpec((1, K_EMB), lambda i: (0, 0))],
        out_specs=[row_spec, pl.BlockSpec((R_TILE, 1), lambda i: (i, 0))],
        out_shape=[jax.ShapeDtypeStruct((R_TOT, c), jnp.float32),
                   jax.ShapeDtypeStruct((R_TOT, 1), jnp.int32)],
    )(e0, o0, e1, o1, wstack, bc2, codebook, csq)

    idx_flat = idx2d[:, 0]
    zq_rows = _sc_gather(idx_flat, codebook)

    zq3 = zq_rows.reshape(n, L_PAD, c)
    zq0 = zq3[:, 0:1024].reshape(n * 1024, c)
    zq1 = zq3[:, 1:1025].reshape(n * 1024, c)
    mstack = jnp.transpose(Wt, (2, 0, 1))
    bt2 = bt[None, :]

    dgrid = (n * 1024) // R_TILE
    xe, xo = pl.pallas_call(
        _dec_body,
        grid=(dgrid,),
        in_specs=[row_spec, row_spec, full3,
                  pl.BlockSpec((1, c), lambda i: (0, 0))],
        out_specs=[row_spec, row_spec],
        out_shape=[jax.ShapeDtypeStruct((n * 1024, c), jnp.float32),
                   jax.ShapeDtypeStruct((n * 1024, c), jnp.float32)],
    )(zq0, zq1, mstack, bt2)

    x_recon = jnp.stack([xe.reshape(n, 1024, c), xo.reshape(n, 1024, c)],
                        axis=2).reshape(n, 2048, c).transpose(0, 2, 1)
    z_e = z_rows.reshape(n, L_PAD, c)[:, :L_OUT].transpose(0, 2, 1)
    z_q = zq3[:, :L_OUT].transpose(0, 2, 1)
    indices = idx_flat.reshape(n, L_PAD)[:, :L_OUT]
    return x_recon, z_e, z_q, indices


# speedup vs baseline: 2.2946x; 2.2946x over previous
import functools

import jax
import jax.numpy as jnp
from jax import lax
from jax.experimental import pallas as pl
from jax.experimental.pallas import tpu as pltpu
from jax.experimental.pallas import tpu_sc as plsc

IN_CH = 64
LAT = 64
K_EMB = 8192
EDIM = 64
L_OUT = 1025
L_PAD = 1056
N_B = 8
R_TOT = N_B * L_PAD
R_TILE = 256
K_TILE = 1024
_PREC = lax.Precision.DEFAULT


def _csq_body(cb_ref, csq_ref):
    cbf = cb_ref[...]
    ones = jnp.ones((1, EDIM), dtype=jnp.float32)
    csq_ref[...] = lax.dot_general(ones, cbf * cbf, (((1,), (1,)), ((), ())),
                                   preferred_element_type=jnp.float32,
                                   precision=lax.Precision.HIGHEST)


def _enc_vq_body(a_ref, b_ref, wa_ref, wb_ref, bc_ref, cb_ref,
                 csq_ref, z_ref, idx_ref):
    zt = jnp.dot(a_ref[...], wa_ref[...], preferred_element_type=jnp.float32,
                 precision=_PREC)
    zt += jnp.dot(b_ref[...], wb_ref[...], preferred_element_type=jnp.float32,
                  precision=_PREC)
    zt += bc_ref[...]
    z_ref[...] = zt

    zt2 = zt * -2.0
    iota = lax.broadcasted_iota(jnp.int32, (R_TILE, K_TILE), 1).astype(jnp.float32)
    big = jnp.float32(1e9)
    best = jnp.full((R_TILE, 1), jnp.inf, dtype=jnp.float32)
    bidx = jnp.zeros((R_TILE, 1), dtype=jnp.float32)
    for kc in range(K_EMB // K_TILE):
        cbk = cb_ref[kc * K_TILE:(kc + 1) * K_TILE, :]
        d = lax.dot_general(zt2, cbk, (((1,), (1,)), ((), ())),
                            preferred_element_type=jnp.float32,
                            precision=_PREC)
        d = d + csq_ref[:, kc * K_TILE:(kc + 1) * K_TILE]
        m = jnp.min(d, axis=1, keepdims=True)
        sel = jnp.where(d == m, iota, big)
        a = jnp.min(sel, axis=1, keepdims=True) + jnp.float32(kc * K_TILE)
        upd = m < best
        best = jnp.where(upd, m, best)
        bidx = jnp.where(upd, a, bidx)
    idx_ref[...] = bidx.astype(jnp.int32)


def _dec_body(zq0_ref, zq1_ref, m_ref, bt_ref, xe_ref, xo_ref):
    a = zq0_ref[...]
    b = zq1_ref[...]
    bt = bt_ref[...]
    xe_ref[...] = (jnp.dot(b, m_ref[0], preferred_element_type=jnp.float32,
                           precision=_PREC)
                   + jnp.dot(a, m_ref[2], preferred_element_type=jnp.float32,
                             precision=_PREC) + bt)
    xo_ref[...] = (jnp.dot(b, m_ref[1], preferred_element_type=jnp.float32,
                           precision=_PREC)
                   + jnp.dot(a, m_ref[3], preferred_element_type=jnp.float32,
                             precision=_PREC) + bt)


def _sc_gather(idx, table):
    info = plsc.get_sparse_core_info()
    nw = info.num_cores * info.num_subcores
    b_per_w = R_TOT // nw
    dpad = 128

    table_p = jnp.pad(table, ((0, 0), (0, dpad - EDIM)))
    mesh = plsc.VectorSubcoreMesh(core_axis_name="c", subcore_axis_name="s")

    @functools.partial(
        pl.kernel, mesh=mesh,
        out_type=jax.ShapeDtypeStruct((R_TOT, dpad), jnp.float32),
        scratch_types=[
            pltpu.VMEM((b_per_w,), jnp.int32),
            pltpu.VMEM((b_per_w, dpad), jnp.float32),
            pltpu.SemaphoreType.DMA,
        ],
    )
    def gather_k(idx_hbm, table_hbm, out_hbm, idx_v, rows_v, sem):
        wid = lax.axis_index("s") * info.num_cores + lax.axis_index("c")
        base = wid * b_per_w
        pltpu.sync_copy(idx_hbm.at[pl.ds(base, b_per_w)], idx_v)
        pltpu.async_copy(table_hbm.at[idx_v], rows_v, sem).wait()
        pltpu.sync_copy(rows_v, out_hbm.at[pl.ds(base, b_per_w)])

    return gather_k(idx, table_p)[:, :EDIM]


def kernel(x, Wc, bc, codebook, Wt, bt):
    n, c, l_in = x.shape

    xT = jnp.transpose(x, (0, 2, 1))
    xp = jnp.pad(xT, ((0, 0), (2, 2 * L_PAD + 2 - l_in - 2), (0, 0)))
    xp2 = xp.reshape(n, L_PAD + 1, 2 * c)
    a_in = xp2[:, :L_PAD].reshape(R_TOT, 2 * c)
    b_in = xp2[:, 1:].reshape(R_TOT, 2 * c)
    wstack = jnp.transpose(Wc, (2, 1, 0))
    wa = jnp.concatenate([wstack[0], wstack[1]], axis=0)
    wb = jnp.concatenate([wstack[2], wstack[3]], axis=0)
    bc2 = bc[None, :]

    csq = pl.pallas_call(
        _csq_body,
        out_shape=jax.ShapeDtypeStruct((1, K_EMB), jnp.float32),
    )(codebook)

    grid = R_TOT // R_TILE
    row_spec = pl.BlockSpec((R_TILE, c), lambda i: (i, 0))
    row_spec2 = pl.BlockSpec((R_TILE, 2 * c), lambda i: (i, 0))
    wfull = pl.BlockSpec((2 * c, c), lambda i: (0, 0))
    z_rows, idx2d = pl.pallas_call(
        _enc_vq_body,
        grid=(grid,),
        in_specs=[row_spec2, row_spec2, wfull, wfull,
                  pl.BlockSpec((1, c), lambda i: (0, 0)),
                  pl.BlockSpec((K_EMB, EDIM), lambda i: (0, 0)),
                  pl.BlockSpec((1, K_EMB), lambda i: (0, 0))],
        out_specs=[row_spec, pl.BlockSpec((R_TILE, 1), lambda i: (i, 0))],
        out_shape=[jax.ShapeDtypeStruct((R_TOT, c), jnp.float32),
                   jax.ShapeDtypeStruct((R_TOT, 1), jnp.int32)],
    )(a_in, b_in, wa, wb, bc2, codebook, csq)

    idx_flat = idx2d[:, 0]
    zq_rows = _sc_gather(idx_flat, codebook)

    zq3 = zq_rows.reshape(n, L_PAD, c)
    zq0 = zq3[:, 0:1024].reshape(n * 1024, c)
    zq1 = zq3[:, 1:1025].reshape(n * 1024, c)
    mstack = jnp.transpose(Wt, (2, 0, 1))
    bt2 = bt[None, :]

    dgrid = (n * 1024) // R_TILE
    full3 = pl.BlockSpec((4, c, c), lambda i: (0, 0, 0))
    xe, xo = pl.pallas_call(
        _dec_body,
        grid=(dgrid,),
        in_specs=[row_spec, row_spec, full3,
                  pl.BlockSpec((1, c), lambda i: (0, 0))],
        out_specs=[row_spec, row_spec],
        out_shape=[jax.ShapeDtypeStruct((n * 1024, c), jnp.float32),
                   jax.ShapeDtypeStruct((n * 1024, c), jnp.float32)],
    )(zq0, zq1, mstack, bt2)

    x_recon = jnp.stack([xe.reshape(n, 1024, c), xo.reshape(n, 1024, c)],
                        axis=2).reshape(n, 2048, c).transpose(0, 2, 1)
    z_e = z_rows.reshape(n, L_PAD, c)[:, :L_OUT].transpose(0, 2, 1)
    z_q = zq3[:, :L_OUT].transpose(0, 2, 1)
    indices = idx_flat.reshape(n, L_PAD)[:, :L_OUT]
    return x_recon, z_e, z_q, indices


# speedup vs baseline: 2.4144x; 1.0522x over previous
import functools

import jax
import jax.numpy as jnp
from jax import lax
from jax.experimental import pallas as pl
from jax.experimental.pallas import tpu as pltpu
from jax.experimental.pallas import tpu_sc as plsc

IN_CH = 64
LAT = 64
K_EMB = 8192
EDIM = 64
L_OUT = 1025
L_PAD = 1056
N_B = 8
R_TOT = N_B * L_PAD
R_TILE = 256
K_TILE = 1024
_PREC = lax.Precision.DEFAULT


def _csq_body(cb_ref, csq_ref):
    cbf = cb_ref[...]
    ones = jnp.ones((1, EDIM), dtype=jnp.float32)
    csq_ref[...] = lax.dot_general(ones, cbf * cbf, (((1,), (1,)), ((), ())),
                                   preferred_element_type=jnp.float32,
                                   precision=lax.Precision.HIGHEST)


def _enc_vq_body(a_ref, b_ref, wa_ref, wb_ref, bc_ref, cb_ref,
                 csq_ref, z_ref, idx_ref):
    zt = jnp.dot(a_ref[...], wa_ref[...], preferred_element_type=jnp.float32,
                 precision=_PREC)
    zt += jnp.dot(b_ref[...], wb_ref[...], preferred_element_type=jnp.float32,
                  precision=_PREC)
    zt += bc_ref[...]
    z_ref[...] = zt

    zt2 = zt * -2.0
    lanes = 128
    nsub = K_TILE // lanes
    ibase = lax.broadcasted_iota(jnp.int32, (R_TILE, lanes), 1).astype(jnp.float32)
    iotas = [ibase + jnp.float32(j * lanes) for j in range(nsub)]
    big = jnp.float32(1e9)
    best = jnp.full((R_TILE, 1), jnp.inf, dtype=jnp.float32)
    bidx = jnp.zeros((R_TILE, 1), dtype=jnp.float32)
    for kc in range(K_EMB // K_TILE):
        cbk = cb_ref[kc * K_TILE:(kc + 1) * K_TILE, :]
        d = lax.dot_general(zt2, cbk, (((1,), (1,)), ((), ())),
                            preferred_element_type=jnp.float32,
                            precision=_PREC)
        v = d[:, 0:lanes] + csq_ref[:, kc * K_TILE:kc * K_TILE + lanes]
        vi = iotas[0]
        for j in range(1, nsub):
            dj = (d[:, j * lanes:(j + 1) * lanes]
                  + csq_ref[:, kc * K_TILE + j * lanes:
                            kc * K_TILE + (j + 1) * lanes])
            upd = dj < v
            v = jnp.where(upd, dj, v)
            vi = jnp.where(upd, iotas[j], vi)
        m = jnp.min(v, axis=1, keepdims=True)
        sel = jnp.where(v == m, vi, big)
        a = jnp.min(sel, axis=1, keepdims=True) + jnp.float32(kc * K_TILE)
        upd = m < best
        best = jnp.where(upd, m, best)
        bidx = jnp.where(upd, a, bidx)
    idx_ref[...] = bidx.astype(jnp.int32)


def _dec_body(zq0_ref, zq1_ref, m_ref, bt_ref, xe_ref, xo_ref):
    a = zq0_ref[...]
    b = zq1_ref[...]
    bt = bt_ref[...]
    xe_ref[...] = (jnp.dot(b, m_ref[0], preferred_element_type=jnp.float32,
                           precision=_PREC)
                   + jnp.dot(a, m_ref[2], preferred_element_type=jnp.float32,
                             precision=_PREC) + bt)
    xo_ref[...] = (jnp.dot(b, m_ref[1], preferred_element_type=jnp.float32,
                           precision=_PREC)
                   + jnp.dot(a, m_ref[3], preferred_element_type=jnp.float32,
                             precision=_PREC) + bt)


def _sc_gather(idx, table):
    info = plsc.get_sparse_core_info()
    nw = info.num_cores * info.num_subcores
    b_per_w = R_TOT // nw
    dpad = 128

    table_p = jnp.pad(table, ((0, 0), (0, dpad - EDIM)))
    mesh = plsc.VectorSubcoreMesh(core_axis_name="c", subcore_axis_name="s")

    @functools.partial(
        pl.kernel, mesh=mesh,
        out_type=jax.ShapeDtypeStruct((R_TOT, dpad), jnp.float32),
        scratch_types=[
            pltpu.VMEM((b_per_w,), jnp.int32),
            pltpu.VMEM((b_per_w, dpad), jnp.float32),
            pltpu.SemaphoreType.DMA,
        ],
    )
    def gather_k(idx_hbm, table_hbm, out_hbm, idx_v, rows_v, sem):
        wid = lax.axis_index("s") * info.num_cores + lax.axis_index("c")
        base = wid * b_per_w
        pltpu.sync_copy(idx_hbm.at[pl.ds(base, b_per_w)], idx_v)
        pltpu.async_copy(table_hbm.at[idx_v], rows_v, sem).wait()
        pltpu.sync_copy(rows_v, out_hbm.at[pl.ds(base, b_per_w)])

    return gather_k(idx, table_p)[:, :EDIM]


def kernel(x, Wc, bc, codebook, Wt, bt):
    n, c, l_in = x.shape

    xT = jnp.transpose(x, (0, 2, 1))
    xp = jnp.pad(xT, ((0, 0), (2, 2 * L_PAD + 2 - l_in - 2), (0, 0)))
    xp2 = xp.reshape(n, L_PAD + 1, 2 * c)
    a_in = xp2[:, :L_PAD].reshape(R_TOT, 2 * c)
    b_in = xp2[:, 1:].reshape(R_TOT, 2 * c)
    wstack = jnp.transpose(Wc, (2, 1, 0))
    wa = jnp.concatenate([wstack[0], wstack[1]], axis=0)
    wb = jnp.concatenate([wstack[2], wstack[3]], axis=0)
    bc2 = bc[None, :]

    csq = pl.pallas_call(
        _csq_body,
        out_shape=jax.ShapeDtypeStruct((1, K_EMB), jnp.float32),
    )(codebook)

    grid = R_TOT // R_TILE
    row_spec = pl.BlockSpec((R_TILE, c), lambda i: (i, 0))
    row_spec2 = pl.BlockSpec((R_TILE, 2 * c), lambda i: (i, 0))
    wfull = pl.BlockSpec((2 * c, c), lambda i: (0, 0))
    z_rows, idx2d = pl.pallas_call(
        _enc_vq_body,
        grid=(grid,),
        in_specs=[row_spec2, row_spec2, wfull, wfull,
                  pl.BlockSpec((1, c), lambda i: (0, 0)),
                  pl.BlockSpec((K_EMB, EDIM), lambda i: (0, 0)),
                  pl.BlockSpec((1, K_EMB), lambda i: (0, 0))],
        out_specs=[row_spec, pl.BlockSpec((R_TILE, 1), lambda i: (i, 0))],
        out_shape=[jax.ShapeDtypeStruct((R_TOT, c), jnp.float32),
                   jax.ShapeDtypeStruct((R_TOT, 1), jnp.int32)],
    )(a_in, b_in, wa, wb, bc2, codebook, csq)

    idx_flat = idx2d[:, 0]
    zq_rows = _sc_gather(idx_flat, codebook)

    zq3 = zq_rows.reshape(n, L_PAD, c)
    zq0 = zq3[:, 0:1024].reshape(n * 1024, c)
    zq1 = zq3[:, 1:1025].reshape(n * 1024, c)
    mstack = jnp.transpose(Wt, (2, 0, 1))
    bt2 = bt[None, :]

    dgrid = (n * 1024) // R_TILE
    full3 = pl.BlockSpec((4, c, c), lambda i: (0, 0, 0))
    xe, xo = pl.pallas_call(
        _dec_body,
        grid=(dgrid,),
        in_specs=[row_spec, row_spec, full3,
                  pl.BlockSpec((1, c), lambda i: (0, 0))],
        out_specs=[row_spec, row_spec],
        out_shape=[jax.ShapeDtypeStruct((n * 1024, c), jnp.float32),
                   jax.ShapeDtypeStruct((n * 1024, c), jnp.float32)],
    )(zq0, zq1, mstack, bt2)

    x_recon = jnp.stack([xe.reshape(n, 1024, c), xo.reshape(n, 1024, c)],
                        axis=2).reshape(n, 2048, c).transpose(0, 2, 1)
    z_e = z_rows.reshape(n, L_PAD, c)[:, :L_OUT].transpose(0, 2, 1)
    z_q = zq3[:, :L_OUT].transpose(0, 2, 1)
    indices = idx_flat.reshape(n, L_PAD)[:, :L_OUT]
    return x_recon, z_e, z_q, indices


# speedup vs baseline: 2.4182x; 1.0016x over previous
import functools

import jax
import jax.numpy as jnp
from jax import lax
from jax.experimental import pallas as pl
from jax.experimental.pallas import tpu as pltpu
from jax.experimental.pallas import tpu_sc as plsc

IN_CH = 64
LAT = 64
K_EMB = 8192
EDIM = 64
L_OUT = 1025
L_PAD = 1056
N_B = 8
R_TOT = N_B * L_PAD
R_TILE = 768
K_TILE = 2048
DEC_TILE = 256
_PREC = lax.Precision.DEFAULT


def _csq_body(cb_ref, csq_ref):
    cbf = cb_ref[...]
    ones = jnp.ones((1, EDIM), dtype=jnp.float32)
    csq_ref[...] = lax.dot_general(ones, cbf * cbf, (((1,), (1,)), ((), ())),
                                   preferred_element_type=jnp.float32,
                                   precision=lax.Precision.HIGHEST)


def _enc_vq_body(a_ref, b_ref, wa_ref, wb_ref, bc_ref, cb_ref,
                 csq_ref, z_ref, idx_ref):
    zt = jnp.dot(a_ref[...], wa_ref[...], preferred_element_type=jnp.float32,
                 precision=_PREC)
    zt += jnp.dot(b_ref[...], wb_ref[...], preferred_element_type=jnp.float32,
                  precision=_PREC)
    zt += bc_ref[...]
    z_ref[...] = zt

    zt2 = zt * -2.0
    lanes = 128
    nsub = K_TILE // lanes
    ibase = lax.broadcasted_iota(jnp.int32, (R_TILE, lanes), 1).astype(jnp.float32)
    iotas = [ibase + jnp.float32(j * lanes) for j in range(nsub)]
    big = jnp.float32(1e9)
    best = jnp.full((R_TILE, 1), jnp.inf, dtype=jnp.float32)
    bidx = jnp.zeros((R_TILE, 1), dtype=jnp.float32)
    for kc in range(K_EMB // K_TILE):
        cbk = cb_ref[kc * K_TILE:(kc + 1) * K_TILE, :]
        d = lax.dot_general(zt2, cbk, (((1,), (1,)), ((), ())),
                            preferred_element_type=jnp.float32,
                            precision=_PREC)
        v = d[:, 0:lanes] + csq_ref[:, kc * K_TILE:kc * K_TILE + lanes]
        vi = iotas[0]
        for j in range(1, nsub):
            dj = (d[:, j * lanes:(j + 1) * lanes]
                  + csq_ref[:, kc * K_TILE + j * lanes:
                            kc * K_TILE + (j + 1) * lanes])
            upd = dj < v
            v = jnp.where(upd, dj, v)
            vi = jnp.where(upd, iotas[j], vi)
        m = jnp.min(v, axis=1, keepdims=True)
        sel = jnp.where(v == m, vi, big)
        a = jnp.min(sel, axis=1, keepdims=True) + jnp.float32(kc * K_TILE)
        upd = m < best
        best = jnp.where(upd, m, best)
        bidx = jnp.where(upd, a, bidx)
    idx_ref[...] = bidx.astype(jnp.int32)


def _dec_body(zq0_ref, zq1_ref, m_ref, bt_ref, xe_ref, xo_ref):
    a = zq0_ref[...]
    b = zq1_ref[...]
    bt = bt_ref[...]
    xe_ref[...] = (jnp.dot(b, m_ref[0], preferred_element_type=jnp.float32,
                           precision=_PREC)
                   + jnp.dot(a, m_ref[2], preferred_element_type=jnp.float32,
                             precision=_PREC) + bt)
    xo_ref[...] = (jnp.dot(b, m_ref[1], preferred_element_type=jnp.float32,
                           precision=_PREC)
                   + jnp.dot(a, m_ref[3], preferred_element_type=jnp.float32,
                             precision=_PREC) + bt)


def _sc_gather(idx, table):
    info = plsc.get_sparse_core_info()
    nw = info.num_cores * info.num_subcores
    b_per_w = R_TOT // nw
    dpad = 128

    table_p = jnp.pad(table, ((0, 0), (0, dpad - EDIM)))
    mesh = plsc.VectorSubcoreMesh(core_axis_name="c", subcore_axis_name="s")

    @functools.partial(
        pl.kernel, mesh=mesh,
        out_type=jax.ShapeDtypeStruct((R_TOT, dpad), jnp.float32),
        scratch_types=[
            pltpu.VMEM((b_per_w,), jnp.int32),
            pltpu.VMEM((b_per_w, dpad), jnp.float32),
            pltpu.SemaphoreType.DMA,
        ],
    )
    def gather_k(idx_hbm, table_hbm, out_hbm, idx_v, rows_v, sem):
        wid = lax.axis_index("s") * info.num_cores + lax.axis_index("c")
        base = wid * b_per_w
        pltpu.sync_copy(idx_hbm.at[pl.ds(base, b_per_w)], idx_v)
        pltpu.async_copy(table_hbm.at[idx_v], rows_v, sem).wait()
        pltpu.sync_copy(rows_v, out_hbm.at[pl.ds(base, b_per_w)])

    return gather_k(idx, table_p)[:, :EDIM]


def kernel(x, Wc, bc, codebook, Wt, bt):
    n, c, l_in = x.shape

    xT = jnp.transpose(x, (0, 2, 1))
    a_in = jnp.pad(xT, ((0, 0), (2, 2 * L_PAD - l_in - 2), (0, 0))
                   ).reshape(R_TOT, 2 * c)
    b_in = jnp.pad(xT, ((0, 0), (0, 2 * L_PAD - l_in), (0, 0))
                   ).reshape(R_TOT, 2 * c)
    wstack = jnp.transpose(Wc, (2, 1, 0))
    wa = jnp.concatenate([wstack[0], wstack[1]], axis=0)
    wb = jnp.concatenate([wstack[2], wstack[3]], axis=0)
    bc2 = bc[None, :]

    csq = pl.pallas_call(
        _csq_body,
        out_shape=jax.ShapeDtypeStruct((1, K_EMB), jnp.float32),
    )(codebook)

    grid = R_TOT // R_TILE
    row_spec = pl.BlockSpec((R_TILE, c), lambda i: (i, 0))
    row_spec2 = pl.BlockSpec((R_TILE, 2 * c), lambda i: (i, 0))
    wfull = pl.BlockSpec((2 * c, c), lambda i: (0, 0))
    z_rows, idx2d = pl.pallas_call(
        _enc_vq_body,
        grid=(grid,),
        in_specs=[row_spec2, row_spec2, wfull, wfull,
                  pl.BlockSpec((1, c), lambda i: (0, 0)),
                  pl.BlockSpec((K_EMB, EDIM), lambda i: (0, 0)),
                  pl.BlockSpec((1, K_EMB), lambda i: (0, 0))],
        out_specs=[row_spec, pl.BlockSpec((R_TILE, 1), lambda i: (i, 0))],
        out_shape=[jax.ShapeDtypeStruct((R_TOT, c), jnp.float32),
                   jax.ShapeDtypeStruct((R_TOT, 1), jnp.int32)],
    )(a_in, b_in, wa, wb, bc2, codebook, csq)

    idx_flat = idx2d[:, 0]
    zq_rows = _sc_gather(idx_flat, codebook)

    zq3 = zq_rows.reshape(n, L_PAD, c)
    zq0 = zq3[:, 0:1024].reshape(n * 1024, c)
    zq1 = zq3[:, 1:1025].reshape(n * 1024, c)
    mstack = jnp.transpose(Wt, (2, 0, 1))
    bt2 = bt[None, :]

    dgrid = (n * 1024) // DEC_TILE
    drow_spec = pl.BlockSpec((DEC_TILE, c), lambda i: (i, 0))
    full3 = pl.BlockSpec((4, c, c), lambda i: (0, 0, 0))
    xe, xo = pl.pallas_call(
        _dec_body,
        grid=(dgrid,),
        in_specs=[drow_spec, drow_spec, full3,
                  pl.BlockSpec((1, c), lambda i: (0, 0))],
        out_specs=[drow_spec, drow_spec],
        out_shape=[jax.ShapeDtypeStruct((n * 1024, c), jnp.float32),
                   jax.ShapeDtypeStruct((n * 1024, c), jnp.float32)],
    )(zq0, zq1, mstack, bt2)

    x_recon = jnp.stack([xe.reshape(n, 1024, c), xo.reshape(n, 1024, c)],
                        axis=2).reshape(n, 2048, c).transpose(0, 2, 1)
    z_e = z_rows.reshape(n, L_PAD, c)[:, :L_OUT].transpose(0, 2, 1)
    z_q = zq3[:, :L_OUT].transpose(0, 2, 1)
    indices = idx_flat.reshape(n, L_PAD)[:, :L_OUT]
    return x_recon, z_e, z_q, indices


# speedup vs baseline: 2.4275x; 1.0039x over previous
import functools

import jax
import jax.numpy as jnp
from jax import lax
from jax.experimental import pallas as pl
from jax.experimental.pallas import tpu as pltpu
from jax.experimental.pallas import tpu_sc as plsc

IN_CH = 64
LAT = 64
K_EMB = 8192
EDIM = 64
L_OUT = 1025
L_PAD = 1056
N_B = 8
R_TOT = N_B * L_PAD
R_TILE = 768
K_TILE = 2048
DEC_TILE = 256
_PREC = lax.Precision.DEFAULT


def _csq_body(cb_ref, csq_ref):
    cbf = cb_ref[...]
    ones = jnp.ones((1, EDIM), dtype=jnp.float32)
    csq_ref[...] = lax.dot_general(ones, cbf * cbf, (((1,), (1,)), ((), ())),
                                   preferred_element_type=jnp.float32,
                                   precision=lax.Precision.HIGHEST)


def _enc_vq_body(a_ref, b_ref, wa_ref, wb_ref, bc_ref, cb_ref,
                 csq_ref, z_ref, idx_ref):
    zt = jnp.dot(a_ref[...], wa_ref[...], preferred_element_type=jnp.float32,
                 precision=_PREC)
    zt += jnp.dot(b_ref[...], wb_ref[...], preferred_element_type=jnp.float32,
                  precision=_PREC)
    zt += bc_ref[...]
    z_ref[...] = zt

    zt2 = zt * -2.0
    lanes = 128
    nsub = K_TILE // lanes
    ibase = lax.broadcasted_iota(jnp.int32, (R_TILE, lanes), 1).astype(jnp.float32)
    iotas = [ibase + jnp.float32(j * lanes) for j in range(nsub)]
    big = jnp.float32(1e9)
    best = jnp.full((R_TILE, 1), jnp.inf, dtype=jnp.float32)
    bidx = jnp.zeros((R_TILE, 1), dtype=jnp.float32)
    for kc in range(K_EMB // K_TILE):
        cbk = cb_ref[kc * K_TILE:(kc + 1) * K_TILE, :]
        d = lax.dot_general(zt2, cbk, (((1,), (1,)), ((), ())),
                            preferred_element_type=jnp.float32,
                            precision=_PREC)
        v = d[:, 0:lanes] + csq_ref[:, kc * K_TILE:kc * K_TILE + lanes]
        vi = iotas[0]
        for j in range(1, nsub):
            dj = (d[:, j * lanes:(j + 1) * lanes]
                  + csq_ref[:, kc * K_TILE + j * lanes:
                            kc * K_TILE + (j + 1) * lanes])
            upd = dj < v
            v = jnp.where(upd, dj, v)
            vi = jnp.where(upd, iotas[j], vi)
        m = jnp.min(v, axis=1, keepdims=True)
        sel = jnp.where(v == m, vi, big)
        a = jnp.min(sel, axis=1, keepdims=True) + jnp.float32(kc * K_TILE)
        upd = m < best
        best = jnp.where(upd, m, best)
        bidx = jnp.where(upd, a, bidx)
    idx_ref[...] = bidx.astype(jnp.int32)


def _dec_body(zq0_ref, zq1_ref, m_ref, bt_ref, xe_ref, xo_ref):
    a = zq0_ref[...]
    b = zq1_ref[...]
    bt = bt_ref[...]
    xe_ref[...] = (jnp.dot(b, m_ref[0], preferred_element_type=jnp.float32,
                           precision=_PREC)
                   + jnp.dot(a, m_ref[2], preferred_element_type=jnp.float32,
                             precision=_PREC) + bt)
    xo_ref[...] = (jnp.dot(b, m_ref[1], preferred_element_type=jnp.float32,
                           precision=_PREC)
                   + jnp.dot(a, m_ref[3], preferred_element_type=jnp.float32,
                             precision=_PREC) + bt)


def _sc_gather(idx, table):
    info = plsc.get_sparse_core_info()
    nw = info.num_cores * info.num_subcores
    b_per_w = R_TOT // nw
    dpad = 128

    table_p = jnp.pad(table, ((0, 0), (0, dpad - EDIM)))
    mesh = plsc.VectorSubcoreMesh(core_axis_name="c", subcore_axis_name="s")

    @functools.partial(
        pl.kernel, mesh=mesh,
        out_type=jax.ShapeDtypeStruct((R_TOT, dpad), jnp.float32),
        scratch_types=[
            pltpu.VMEM((b_per_w,), jnp.int32),
            pltpu.VMEM((b_per_w, dpad), jnp.float32),
            pltpu.SemaphoreType.DMA,
        ],
    )
    def gather_k(idx_hbm, table_hbm, out_hbm, idx_v, rows_v, sem):
        wid = lax.axis_index("s") * info.num_cores + lax.axis_index("c")
        base = wid * b_per_w
        pltpu.sync_copy(idx_hbm.at[pl.ds(base, b_per_w)], idx_v)
        nstream = 3
        per = b_per_w // nstream
        handles = [
            pltpu.async_copy(table_hbm.at[idx_v.at[pl.ds(j * per, per)]],
                             rows_v.at[pl.ds(j * per, per)], sem)
            for j in range(nstream)
        ]
        for h in handles:
            h.wait()
        pltpu.sync_copy(rows_v, out_hbm.at[pl.ds(base, b_per_w)])

    return gather_k(idx, table_p)[:, :EDIM]


def kernel(x, Wc, bc, codebook, Wt, bt):
    n, c, l_in = x.shape

    xT = jnp.transpose(x, (0, 2, 1))
    a_in = jnp.pad(xT, ((0, 0), (2, 2 * L_PAD - l_in - 2), (0, 0))
                   ).reshape(R_TOT, 2 * c)
    b_in = jnp.pad(xT, ((0, 0), (0, 2 * L_PAD - l_in), (0, 0))
                   ).reshape(R_TOT, 2 * c)
    wstack = jnp.transpose(Wc, (2, 1, 0))
    wa = jnp.concatenate([wstack[0], wstack[1]], axis=0)
    wb = jnp.concatenate([wstack[2], wstack[3]], axis=0)
    bc2 = bc[None, :]

    csq = pl.pallas_call(
        _csq_body,
        out_shape=jax.ShapeDtypeStruct((1, K_EMB), jnp.float32),
    )(codebook)

    grid = R_TOT // R_TILE
    row_spec = pl.BlockSpec((R_TILE, c), lambda i: (i, 0))
    row_spec2 = pl.BlockSpec((R_TILE, 2 * c), lambda i: (i, 0))
    wfull = pl.BlockSpec((2 * c, c), lambda i: (0, 0))
    z_rows, idx2d = pl.pallas_call(
        _enc_vq_body,
        grid=(grid,),
        in_specs=[row_spec2, row_spec2, wfull, wfull,
                  pl.BlockSpec((1, c), lambda i: (0, 0)),
                  pl.BlockSpec((K_EMB, EDIM), lambda i: (0, 0)),
                  pl.BlockSpec((1, K_EMB), lambda i: (0, 0))],
        out_specs=[row_spec, pl.BlockSpec((R_TILE, 1), lambda i: (i, 0))],
        out_shape=[jax.ShapeDtypeStruct((R_TOT, c), jnp.float32),
                   jax.ShapeDtypeStruct((R_TOT, 1), jnp.int32)],
    )(a_in, b_in, wa, wb, bc2, codebook, csq)

    idx_flat = idx2d[:, 0]
    zq_rows = _sc_gather(idx_flat, codebook)

    zq3 = zq_rows.reshape(n, L_PAD, c)
    zq0 = zq3[:, 0:1024].reshape(n * 1024, c)
    zq1 = zq3[:, 1:1025].reshape(n * 1024, c)
    mstack = jnp.transpose(Wt, (2, 0, 1))
    bt2 = bt[None, :]

    dgrid = (n * 1024) // DEC_TILE
    drow_spec = pl.BlockSpec((DEC_TILE, c), lambda i: (i, 0))
    full3 = pl.BlockSpec((4, c, c), lambda i: (0, 0, 0))
    xe, xo = pl.pallas_call(
        _dec_body,
        grid=(dgrid,),
        in_specs=[drow_spec, drow_spec, full3,
                  pl.BlockSpec((1, c), lambda i: (0, 0))],
        out_specs=[drow_spec, drow_spec],
        out_shape=[jax.ShapeDtypeStruct((n * 1024, c), jnp.float32),
                   jax.ShapeDtypeStruct((n * 1024, c), jnp.float32)],
    )(zq0, zq1, mstack, bt2)

    x_recon = jnp.stack([xe.reshape(n, 1024, c), xo.reshape(n, 1024, c)],
                        axis=2).reshape(n, 2048, c).transpose(0, 2, 1)
    z_e = z_rows.reshape(n, L_PAD, c)[:, :L_OUT].transpose(0, 2, 1)
    z_q = zq3[:, :L_OUT].transpose(0, 2, 1)
    indices = idx_flat.reshape(n, L_PAD)[:, :L_OUT]
    return x_recon, z_e, z_q, indices


# speedup vs baseline: 2.6630x; 1.0970x over previous
import functools

import jax
import jax.numpy as jnp
from jax import lax
from jax.experimental import pallas as pl
from jax.experimental.pallas import tpu as pltpu
from jax.experimental.pallas import tpu_sc as plsc

IN_CH = 64
LAT = 64
K_EMB = 8192
EDIM = 64
L_OUT = 1025
L_PAD = 1056
N_B = 8
R_TOT = N_B * L_PAD
R_TILE = 768
K_TILE = 2048
DEC_TILE = 256
_PREC = lax.Precision.DEFAULT


def _csq_body(cb_ref, csq_ref):
    cbf = cb_ref[...]
    ones = jnp.ones((1, EDIM), dtype=jnp.float32)
    csq_ref[...] = lax.dot_general(ones, cbf * cbf, (((1,), (1,)), ((), ())),
                                   preferred_element_type=jnp.float32,
                                   precision=lax.Precision.HIGHEST)


def _enc_vq_body(a_ref, b_ref, wa_ref, wb_ref, bc_ref, cb_ref,
                 csq_ref, z_ref, idx_ref):
    zt = jnp.dot(a_ref[...], wa_ref[...], preferred_element_type=jnp.float32,
                 precision=_PREC)
    zt += jnp.dot(b_ref[...], wb_ref[...], preferred_element_type=jnp.float32,
                  precision=_PREC)
    zt += bc_ref[...]
    z_ref[...] = zt

    zt2 = zt * -2.0
    lanes = 128
    nsub = K_TILE // lanes
    ibase = lax.broadcasted_iota(jnp.int32, (R_TILE, lanes), 1).astype(jnp.float32)
    iotas = [ibase + jnp.float32(j * lanes) for j in range(nsub)]
    big = jnp.float32(1e9)
    best = jnp.full((R_TILE, 1), jnp.inf, dtype=jnp.float32)
    bidx = jnp.zeros((R_TILE, 1), dtype=jnp.float32)
    for kc in range(K_EMB // K_TILE):
        cbk = cb_ref[kc * K_TILE:(kc + 1) * K_TILE, :]
        d = lax.dot_general(zt2, cbk, (((1,), (1,)), ((), ())),
                            preferred_element_type=jnp.float32,
                            precision=_PREC)
        v = d[:, 0:lanes] + csq_ref[:, kc * K_TILE:kc * K_TILE + lanes]
        vi = iotas[0]
        for j in range(1, nsub):
            dj = (d[:, j * lanes:(j + 1) * lanes]
                  + csq_ref[:, kc * K_TILE + j * lanes:
                            kc * K_TILE + (j + 1) * lanes])
            upd = dj < v
            v = jnp.where(upd, dj, v)
            vi = jnp.where(upd, iotas[j], vi)
        m = jnp.min(v, axis=1, keepdims=True)
        sel = jnp.where(v == m, vi, big)
        a = jnp.min(sel, axis=1, keepdims=True) + jnp.float32(kc * K_TILE)
        upd = m < best
        best = jnp.where(upd, m, best)
        bidx = jnp.where(upd, a, bidx)
    idx_ref[...] = bidx.astype(jnp.int32)


def _dec_body(zq_ref, m_ref, bt_ref, xe_ref, xo_ref):
    blk = zq_ref[...]
    a = blk[0:1024]
    b = blk[1:1025]
    bt = bt_ref[...]
    xe_ref[...] = (jnp.dot(b, m_ref[0], preferred_element_type=jnp.float32,
                           precision=_PREC)
                   + jnp.dot(a, m_ref[2], preferred_element_type=jnp.float32,
                             precision=_PREC) + bt)
    xo_ref[...] = (jnp.dot(b, m_ref[1], preferred_element_type=jnp.float32,
                           precision=_PREC)
                   + jnp.dot(a, m_ref[3], preferred_element_type=jnp.float32,
                             precision=_PREC) + bt)


def _sc_gather(idx, table):
    info = plsc.get_sparse_core_info()
    nw = info.num_cores * info.num_subcores
    b_per_w = R_TOT // nw
    dpad = 128

    table_p = jnp.pad(table, ((0, 0), (0, dpad - EDIM)))
    mesh = plsc.VectorSubcoreMesh(core_axis_name="c", subcore_axis_name="s")

    @functools.partial(
        pl.kernel, mesh=mesh,
        out_type=jax.ShapeDtypeStruct((R_TOT, dpad), jnp.float32),
        scratch_types=[
            pltpu.VMEM((b_per_w,), jnp.int32),
            pltpu.VMEM((b_per_w, dpad), jnp.float32),
            pltpu.SemaphoreType.DMA,
        ],
    )
    def gather_k(idx_hbm, table_hbm, out_hbm, idx_v, rows_v, sem):
        wid = lax.axis_index("s") * info.num_cores + lax.axis_index("c")
        base = wid * b_per_w
        pltpu.sync_copy(idx_hbm.at[pl.ds(base, b_per_w)], idx_v)
        pltpu.async_copy(table_hbm.at[idx_v], rows_v, sem).wait()
        pltpu.sync_copy(rows_v, out_hbm.at[pl.ds(base, b_per_w)])

    return gather_k(idx, table_p)[:, :EDIM]


def kernel(x, Wc, bc, codebook, Wt, bt):
    n, c, l_in = x.shape

    xT = jnp.transpose(x, (0, 2, 1))
    a_in = jnp.pad(xT, ((0, 0), (2, 2 * L_PAD - l_in - 2), (0, 0))
                   ).reshape(R_TOT, 2 * c)
    b_in = jnp.pad(xT, ((0, 0), (0, 2 * L_PAD - l_in), (0, 0))
                   ).reshape(R_TOT, 2 * c)
    wstack = jnp.transpose(Wc, (2, 1, 0))
    wa = jnp.concatenate([wstack[0], wstack[1]], axis=0)
    wb = jnp.concatenate([wstack[2], wstack[3]], axis=0)
    bc2 = bc[None, :]

    csq = pl.pallas_call(
        _csq_body,
        out_shape=jax.ShapeDtypeStruct((1, K_EMB), jnp.float32),
    )(codebook)

    grid = R_TOT // R_TILE
    row_spec = pl.BlockSpec((R_TILE, c), lambda i: (i, 0))
    row_spec2 = pl.BlockSpec((R_TILE, 2 * c), lambda i: (i, 0))
    wfull = pl.BlockSpec((2 * c, c), lambda i: (0, 0))
    z_rows, idx2d = pl.pallas_call(
        _enc_vq_body,
        grid=(grid,),
        in_specs=[row_spec2, row_spec2, wfull, wfull,
                  pl.BlockSpec((1, c), lambda i: (0, 0)),
                  pl.BlockSpec((K_EMB, EDIM), lambda i: (0, 0)),
                  pl.BlockSpec((1, K_EMB), lambda i: (0, 0))],
        out_specs=[row_spec, pl.BlockSpec((R_TILE, 1), lambda i: (i, 0))],
        out_shape=[jax.ShapeDtypeStruct((R_TOT, c), jnp.float32),
                   jax.ShapeDtypeStruct((R_TOT, 1), jnp.int32)],
    )(a_in, b_in, wa, wb, bc2, codebook, csq)

    idx_flat = idx2d[:, 0]
    zq_rows = _sc_gather(idx_flat, codebook)

    zq3 = zq_rows.reshape(n, L_PAD, c)
    mstack = jnp.transpose(Wt, (2, 0, 1))
    bt2 = bt[None, :]

    full3 = pl.BlockSpec((4, c, c), lambda i: (0, 0, 0))
    xe, xo = pl.pallas_call(
        _dec_body,
        grid=(n,),
        in_specs=[pl.BlockSpec((L_PAD, c), lambda i: (i, 0)), full3,
                  pl.BlockSpec((1, c), lambda i: (0, 0))],
        out_specs=[pl.BlockSpec((1024, c), lambda i: (i, 0)),
                   pl.BlockSpec((1024, c), lambda i: (i, 0))],
        out_shape=[jax.ShapeDtypeStruct((n * 1024, c), jnp.float32),
                   jax.ShapeDtypeStruct((n * 1024, c), jnp.float32)],
    )(zq_rows, mstack, bt2)

    x_recon = jnp.stack([xe.reshape(n, 1024, c), xo.reshape(n, 1024, c)],
                        axis=2).reshape(n, 2048, c).transpose(0, 2, 1)
    z_e = z_rows.reshape(n, L_PAD, c)[:, :L_OUT].transpose(0, 2, 1)
    z_q = zq3[:, :L_OUT].transpose(0, 2, 1)
    indices = idx_flat.reshape(n, L_PAD)[:, :L_OUT]
    return x_recon, z_e, z_q, indices
